# Initial kernel scaffold; baseline (speedup 1.0000x reference)
import jax
import jax.numpy as jnp
from jax.experimental import pallas as pl


def kernel(x, alpha, beta, clss):
    raise NotImplementedError("write your pallas kernel here")


# speedup vs baseline: 1.7246x; 1.7246x over previous
import jax
import jax.numpy as jnp
from jax.experimental import pallas as pl

_BATCH_BLOCK = 1024


def _bias_kern(x_ref, ab_ref, clss_ref, o_ref):
    n = x_ref.shape[1]
    cols = jax.lax.broadcasted_iota(jnp.int32, (1, n), 1)
    c = clss_ref[...].reshape(clss_ref.shape[1], 1)
    mask = jnp.any(c == cols, axis=0, keepdims=True)
    alpha = ab_ref[0, 0]
    beta = ab_ref[0, 1]
    a = jnp.where(mask, alpha, jnp.float32(1.0))
    b = jnp.where(mask, beta, jnp.float32(0.0))
    o_ref[...] = x_ref[...] * a + b


def kernel(x, alpha, beta, clss):
    batch, n = x.shape
    ab = jnp.stack([alpha[0], beta[0]]).reshape(1, 2).astype(jnp.float32)
    clss2 = clss.astype(jnp.int32).reshape(1, -1)
    nb = batch // _BATCH_BLOCK
    return pl.pallas_call(
        _bias_kern,
        grid=(nb,),
        in_specs=[
            pl.BlockSpec((_BATCH_BLOCK, n), lambda i: (i, 0)),
            pl.BlockSpec((1, 2), lambda i: (0, 0)),
            pl.BlockSpec(clss2.shape, lambda i: (0, 0)),
        ],
        out_specs=pl.BlockSpec((_BATCH_BLOCK, n), lambda i: (i, 0)),
        out_shape=jax.ShapeDtypeStruct((batch, n), x.dtype),
    )(x, ab, clss2)


# speedup vs baseline: 1.7281x; 1.0020x over previous
import jax
import jax.numpy as jnp
from jax.experimental import pallas as pl

_BATCH_BLOCK = 2048


def _bias_kern(x_ref, ab_ref, clss_ref, o_ref):
    n = x_ref.shape[1]
    cols = jax.lax.broadcasted_iota(jnp.int32, (1, n), 1)
    c = clss_ref[...].reshape(clss_ref.shape[1], 1)
    mask = jnp.any(c == cols, axis=0, keepdims=True)
    alpha = ab_ref[0, 0]
    beta = ab_ref[0, 1]
    a = jnp.where(mask, alpha, jnp.float32(1.0))
    b = jnp.where(mask, beta, jnp.float32(0.0))
    o_ref[...] = x_ref[...] * a + b


def kernel(x, alpha, beta, clss):
    batch, n = x.shape
    ab = jnp.stack([alpha[0], beta[0]]).reshape(1, 2).astype(jnp.float32)
    clss2 = clss.astype(jnp.int32).reshape(1, -1)
    nb = batch // _BATCH_BLOCK
    return pl.pallas_call(
        _bias_kern,
        grid=(nb,),
        in_specs=[
            pl.BlockSpec((_BATCH_BLOCK, n), lambda i: (i, 0)),
            pl.BlockSpec((1, 2), lambda i: (0, 0)),
            pl.BlockSpec(clss2.shape, lambda i: (0, 0)),
        ],
        out_specs=pl.BlockSpec((_BATCH_BLOCK, n), lambda i: (i, 0)),
        out_shape=jax.ShapeDtypeStruct((batch, n), x.dtype),
    )(x, ab, clss2)


# speedup vs baseline: 1.7292x; 1.0006x over previous
import jax
import jax.numpy as jnp
from jax.experimental import pallas as pl
from jax.experimental.pallas import tpu as pltpu

_CHUNK = 512
_DEPTH = 8


def _bias_kern(x_hbm, ab_ref, clss_ref, o_hbm, inbuf, outbuf, insem, outsem):
    nrows, n = x_hbm.shape
    nchunks = nrows // _CHUNK

    cols = jax.lax.broadcasted_iota(jnp.int32, (1, n), 1)
    c = clss_ref[...].reshape(clss_ref.shape[1], 1)
    mask = jnp.any(c == cols, axis=0, keepdims=True)
    a = jnp.where(mask, ab_ref[0, 0], jnp.float32(1.0))
    b = jnp.where(mask, ab_ref[0, 1], jnp.float32(0.0))

    def in_copy(i, slot):
        return pltpu.make_async_copy(
            x_hbm.at[pl.ds(i * _CHUNK, _CHUNK), :], inbuf.at[slot],
            insem.at[slot])

    def out_copy(i, slot):
        return pltpu.make_async_copy(
            outbuf.at[slot], o_hbm.at[pl.ds(i * _CHUNK, _CHUNK), :],
            outsem.at[slot])

    for s in range(_DEPTH):
        in_copy(s, s).start()

    def body(i, carry):
        slot = jax.lax.rem(i, _DEPTH)
        in_copy(i, slot).wait()

        @pl.when(i >= _DEPTH)
        def _():
            out_copy(i - _DEPTH, slot).wait()

        outbuf[slot] = inbuf[slot] * a + b
        out_copy(i, slot).start()

        @pl.when(i + _DEPTH < nchunks)
        def _():
            in_copy(i + _DEPTH, slot).start()

        return carry

    jax.lax.fori_loop(0, nchunks, body, 0)

    def drain(i, carry):
        out_copy(i, jax.lax.rem(i, _DEPTH)).wait()
        return carry

    jax.lax.fori_loop(nchunks - _DEPTH, nchunks, drain, 0)


def kernel(x, alpha, beta, clss):
    batch, n = x.shape
    ab = jnp.stack([alpha[0], beta[0]]).reshape(1, 2).astype(jnp.float32)
    clss2 = clss.astype(jnp.int32).reshape(1, -1)
    return pl.pallas_call(
        _bias_kern,
        in_specs=[
            pl.BlockSpec(memory_space=pl.ANY),
            pl.BlockSpec(memory_space=pltpu.VMEM),
            pl.BlockSpec(memory_space=pltpu.VMEM),
        ],
        out_specs=pl.BlockSpec(memory_space=pl.ANY),
        out_shape=jax.ShapeDtypeStruct((batch, n), x.dtype),
        scratch_shapes=[
            pltpu.VMEM((_DEPTH, _CHUNK, n), jnp.float32),
            pltpu.VMEM((_DEPTH, _CHUNK, n), jnp.float32),
            pltpu.SemaphoreType.DMA((_DEPTH,)),
            pltpu.SemaphoreType.DMA((_DEPTH,)),
        ],
    )(x, ab, clss2)


# speedup vs baseline: 2.2128x; 1.2797x over previous
import jax
import jax.numpy as jnp
from jax import lax
from jax.experimental import pallas as pl
from jax.experimental.pallas import tpu as pltpu

_TILE = 128
_ROWCH = 2048
_DEPTH = 4


def _fix_kern(x_hbm, ab_ref, clss_ref, o_hbm, buf, tbuf, insem, outsem,
              tinsem, toutsem):
    del x_hbm
    nrows, n = o_hbm.shape
    nr = nrows // _ROWCH
    n_full = n // _TILE
    tail_w = n - n_full * _TILE
    c = clss_ref[...]
    lo = jnp.min(c)
    hi = jnp.max(c)
    t_lo = lo // _TILE
    t_hi = hi // _TILE
    alpha = ab_ref[0, 0]
    beta = ab_ref[0, 1]
    cT = c.reshape(c.shape[1], 1)

    def ab_row(col0, width):
        cols = col0 + jax.lax.broadcasted_iota(jnp.int32, (1, width), 1)
        m = jnp.any(cT == cols, axis=0, keepdims=True)
        a = jnp.where(m, alpha, jnp.float32(1.0))
        b = jnp.where(m, beta, jnp.float32(0.0))
        return a, b

    nft = jnp.maximum(jnp.minimum(t_hi, n_full - 1) - t_lo + 1, 0)
    k1 = nft * nr

    def col0_of(k):
        return (t_lo + k // nr) * _TILE

    def row0_of(k):
        return lax.rem(k, nr) * _ROWCH

    def in_copy(k, slot):
        return pltpu.make_async_copy(
            o_hbm.at[pl.ds(row0_of(k), _ROWCH), pl.ds(col0_of(k), _TILE)],
            buf.at[slot], insem.at[slot])

    def out_copy(k, slot):
        return pltpu.make_async_copy(
            buf.at[slot],
            o_hbm.at[pl.ds(row0_of(k), _ROWCH), pl.ds(col0_of(k), _TILE)],
            outsem.at[slot])

    def warm1(k, carry):
        in_copy(k, lax.rem(k, _DEPTH)).start()
        return carry

    lax.fori_loop(0, jnp.minimum(k1, _DEPTH), warm1, 0)

    def body1(k, carry):
        slot = lax.rem(k, _DEPTH)
        in_copy(k, slot).wait()

        @pl.when(k >= _DEPTH)
        def _():
            out_copy(k - _DEPTH, slot).wait()

        a, b = ab_row(col0_of(k), _TILE)
        buf[slot] = buf[slot] * a + b
        out_copy(k, slot).start()

        @pl.when(k + _DEPTH < k1)
        def _():
            in_copy(k + _DEPTH, slot).start()

        return carry

    lax.fori_loop(0, k1, body1, 0)

    def drain1(k, carry):
        out_copy(k, lax.rem(k, _DEPTH)).wait()
        return carry

    lax.fori_loop(jnp.maximum(k1 - _DEPTH, 0), k1, drain1, 0)

    col_t = n_full * _TILE
    kt = jnp.where(t_hi >= n_full, nr, 0)

    def tin_copy(k, slot):
        return pltpu.make_async_copy(
            o_hbm.at[pl.ds(k * _ROWCH, _ROWCH), pl.ds(col_t, tail_w)],
            tbuf.at[slot], tinsem.at[slot])

    def tout_copy(k, slot):
        return pltpu.make_async_copy(
            tbuf.at[slot],
            o_hbm.at[pl.ds(k * _ROWCH, _ROWCH), pl.ds(col_t, tail_w)],
            toutsem.at[slot])

    def warm2(k, carry):
        tin_copy(k, lax.rem(k, _DEPTH)).start()
        return carry

    lax.fori_loop(0, jnp.minimum(kt, _DEPTH), warm2, 0)

    def body2(k, carry):
        slot = lax.rem(k, _DEPTH)
        tin_copy(k, slot).wait()

        @pl.when(k >= _DEPTH)
        def _():
            tout_copy(k - _DEPTH, slot).wait()

        a, b = ab_row(col_t, tail_w)
        tbuf[slot] = tbuf[slot] * a + b
        tout_copy(k, slot).start()

        @pl.when(k + _DEPTH < kt)
        def _():
            tin_copy(k + _DEPTH, slot).start()

        return carry

    lax.fori_loop(0, kt, body2, 0)

    def drain2(k, carry):
        tout_copy(k, lax.rem(k, _DEPTH)).wait()
        return carry

    lax.fori_loop(jnp.maximum(kt - _DEPTH, 0), kt, drain2, 0)


def kernel(x, alpha, beta, clss):
    batch, n = x.shape
    ab = jnp.stack([alpha[0], beta[0]]).reshape(1, 2).astype(jnp.float32)
    clss2 = clss.astype(jnp.int32).reshape(1, -1)
    tail_w = n - (n // _TILE) * _TILE
    return pl.pallas_call(
        _fix_kern,
        in_specs=[
            pl.BlockSpec(memory_space=pl.ANY),
            pl.BlockSpec(memory_space=pltpu.VMEM),
            pl.BlockSpec(memory_space=pltpu.VMEM),
        ],
        out_specs=pl.BlockSpec(memory_space=pl.ANY),
        out_shape=jax.ShapeDtypeStruct((batch, n), x.dtype),
        input_output_aliases={0: 0},
        scratch_shapes=[
            pltpu.VMEM((_DEPTH, _ROWCH, _TILE), jnp.float32),
            pltpu.VMEM((_DEPTH, _ROWCH, tail_w), jnp.float32),
            pltpu.SemaphoreType.DMA((_DEPTH,)),
            pltpu.SemaphoreType.DMA((_DEPTH,)),
            pltpu.SemaphoreType.DMA((_DEPTH,)),
            pltpu.SemaphoreType.DMA((_DEPTH,)),
        ],
    )(x, ab, clss2)
